# Initial kernel scaffold; baseline (speedup 1.0000x reference)
import jax
import jax.numpy as jnp
from jax.experimental import pallas as pl


def kernel(x, Embed):
    raise NotImplementedError("write your pallas kernel here")


# speedup vs baseline: 3.7032x; 3.7032x over previous
import functools

import jax
import jax.numpy as jnp
from jax import lax
from jax.experimental import pallas as pl
from jax.experimental.pallas import tpu as pltpu
from jax.experimental.pallas import tpu_sc as plsc


def _make_sc_broadcast(B, L, D, dtype):
    info = plsc.get_sparse_core_info()
    NC, NS = info.num_cores, info.num_subcores
    NW = NC * NS
    assert L % NW == 0
    rows_per_w = L // NW
    chunk = min(rows_per_w, 64)
    n_chunks = rows_per_w // chunk
    mesh = plsc.VectorSubcoreMesh(core_axis_name="c", subcore_axis_name="s")

    @functools.partial(
        pl.kernel,
        mesh=mesh,
        out_type=jax.ShapeDtypeStruct((B, L, D), dtype),
        scratch_types=[pltpu.VMEM((chunk, D), dtype)],
    )
    def k(emb_hbm, out_hbm, buf):
        wid = lax.axis_index("s") * NC + lax.axis_index("c")
        base = wid * rows_per_w
        for i in range(n_chunks):
            r0 = base + i * chunk
            pltpu.sync_copy(emb_hbm.at[pl.ds(r0, chunk)], buf)
            for b in range(B):
                pltpu.sync_copy(buf, out_hbm.at[b, pl.ds(r0, chunk)])

    return k


def kernel(x, Embed):
    B, L, D = x.shape
    k = _make_sc_broadcast(B, L, D, Embed.dtype)
    return k(Embed)
